# baseline (device time: 207983 ns/iter reference)
import functools

import jax
import jax.numpy as jnp
from jax import lax
from jax.experimental import pallas as pl
from jax.experimental.pallas import tpu as pltpu

N_DEV = 8
N_HOPS = 4


def kernel(x, w_mat):
    x = x.astype(jnp.bfloat16)
    m_per, k = x.shape
    _, n_loc = w_mat.shape
    half = m_per // 2
    wcols = n_loc // 8

    def body(x_ref, w_hbm_ref, out_ref, cw_ref, ccw_ref, cw_half_ref,
             ccw_half_ref, w_ref, w_stage, amax_src, amax_g1, amax_g2,
             cw_send_sems, cw_recv_sems, ccw_send_sems, ccw_recv_sems,
             amax_send_sems, a1_recv_sems, a2_recv_sems, w_sem):
        my = lax.axis_index("i")
        left = lax.rem(my + N_DEV - 1, N_DEV)
        right = lax.rem(my + 1, N_DEV)

        barrier_sem = pltpu.get_barrier_semaphore()
        for nbr in (left, right):
            pl.semaphore_signal(barrier_sem, inc=1, device_id=(nbr,),
                                device_id_type=pl.DeviceIdType.MESH)
        pl.semaphore_wait(barrier_sem, 2)

        def copy(src, dst, ssem, rsem, tgt):
            return pltpu.make_async_remote_copy(
                src_ref=src, dst_ref=dst, send_sem=ssem, recv_sem=rsem,
                device_id=(tgt,), device_id_type=pl.DeviceIdType.MESH)

        def chunk_gemm(chunk, row0, nrows, amax):
            y = jnp.dot(chunk, w_ref[...], preferred_element_type=jnp.float32)
            y = jnp.maximum(y, 0.0)
            out_ref[pl.ds(row0, nrows), :] = y
            return jnp.maximum(amax, jnp.max(y))

        cw = [copy(x_ref, cw_ref.at[0], cw_send_sems.at[0],
                   cw_recv_sems.at[0], right)]
        ccw = [copy(x_ref, ccw_ref.at[0], ccw_send_sems.at[0],
                    ccw_recv_sems.at[0], left)]
        cw[0].start()
        ccw[0].start()

        for t in range(n_loc // wcols):
            wdma = pltpu.make_async_copy(
                w_hbm_ref.at[:, pl.ds(t * wcols, wcols)], w_stage, w_sem)
            wdma.start()
            wdma.wait()
            rows = k // 4
            for r in range(4):
                w_ref[pl.ds(r * rows, rows), pl.ds(t * wcols, wcols)] = (
                    w_stage[pl.ds(r * rows, rows), :].astype(jnp.bfloat16))

        amax = chunk_gemm(x_ref[...], my * m_per, m_per, jnp.float32(0.0))

        for h in range(N_HOPS - 1):
            cw[h].wait_recv()
            if h < N_HOPS - 2:
                nxt = copy(cw_ref.at[h], cw_ref.at[h + 1],
                           cw_send_sems.at[h + 1], cw_recv_sems.at[h + 1],
                           right)
            else:
                nxt = copy(cw_ref.at[h, pl.ds(0, half)],
                           cw_half_ref,
                           cw_send_sems.at[h + 1], cw_recv_sems.at[h + 1],
                           right)
            nxt.start()
            cw.append(nxt)

            ccw[h].wait_recv()
            if h < N_HOPS - 2:
                nxt = copy(ccw_ref.at[h], ccw_ref.at[h + 1],
                           ccw_send_sems.at[h + 1], ccw_recv_sems.at[h + 1],
                           left)
            else:
                nxt = copy(ccw_ref.at[h, pl.ds(half, half)],
                           ccw_half_ref,
                           ccw_send_sems.at[h + 1], ccw_recv_sems.at[h + 1],
                           left)
            nxt.start()
            ccw.append(nxt)

            cw_origin = lax.rem(my + N_DEV - (h + 1), N_DEV)
            ccw_origin = lax.rem(my + h + 1, N_DEV)
            amax = chunk_gemm(cw_ref[h], cw_origin * m_per, m_per, amax)
            amax = chunk_gemm(ccw_ref[h], ccw_origin * m_per, m_per, amax)

        def amax_exchange(val, src_slot, gather, send_base, recv_sems):
            amax_src[src_slot] = jnp.full((1, 128), val, jnp.float32)
            gather[0] = amax_src[src_slot]
            rs = []
            for d in range(1, N_DEV):
                r = copy(amax_src.at[src_slot], gather.at[d],
                         amax_send_sems.at[send_base + d - 1],
                         recv_sems.at[d], lax.rem(my + d, N_DEV))
                r.start()
                rs.append(r)
            return rs

        def quant_block(row0, nrows, inv, scale):
            blk = out_ref[pl.ds(row0, nrows), :]
            q = (blk * inv).astype(jnp.float8_e4m3fn)
            out_ref[pl.ds(row0, nrows), :] = q.astype(jnp.float32) * scale

        far = lax.rem(my + N_DEV // 2, N_DEV)

        round1 = amax_exchange(amax, 0, amax_g1, 0, a1_recv_sems)
        for r in round1:
            r.wait_recv()
        g1 = jnp.max(amax_g1[...])
        inv1 = 448.0 / g1
        scale1 = g1 / 448.0
        for b in range(N_DEV):
            @pl.when(b != far)
            def _():
                quant_block(b * m_per, half, inv1, scale1)
                quant_block(b * m_per + half, half, inv1, scale1)

        cw[N_HOPS - 1].wait_recv()
        t1 = chunk_gemm(cw_half_ref[...], far * m_per, half, jnp.float32(0.0))
        ccw[N_HOPS - 1].wait_recv()
        tail = chunk_gemm(ccw_half_ref[...], far * m_per + half, half, t1)

        round2 = amax_exchange(tail, 1, amax_g2, N_DEV - 1, a2_recv_sems)
        for r in round2:
            r.wait_recv()
        g2 = jnp.maximum(g1, jnp.max(amax_g2[...]))
        inv2 = 448.0 / g2
        scale2 = g2 / 448.0
        quant_block(far * m_per, half, inv2, scale2)
        quant_block(far * m_per + half, half, inv2, scale2)
        for b in range(N_DEV):
            @pl.when(jnp.logical_and(g2 > g1, b != far))
            def _():
                quant_block(b * m_per, half, inv2, scale2)
                quant_block(b * m_per + half, half, inv2, scale2)

        for r in cw + ccw + round1 + round2:
            r.wait_send()

        @functools.partial(pl.run_scoped, ack=pltpu.SemaphoreType.REGULAR)
        def _(ack):
            for d in range(1, N_DEV):
                pl.semaphore_signal(ack, inc=1,
                                    device_id=(lax.rem(my + d, N_DEV),),
                                    device_id_type=pl.DeviceIdType.MESH)
            pl.semaphore_wait(ack, N_DEV - 1)

    return pl.pallas_call(
        body,
        out_shape=jax.ShapeDtypeStruct((N_DEV * m_per, n_loc), jnp.float32),
        in_specs=[
            pl.BlockSpec(memory_space=pltpu.VMEM),
            pl.BlockSpec(memory_space=pl.ANY),
        ],
        out_specs=pl.BlockSpec(memory_space=pltpu.VMEM),
        scratch_shapes=[
            pltpu.VMEM((N_HOPS - 1, m_per, k), x.dtype),
            pltpu.VMEM((N_HOPS - 1, m_per, k), x.dtype),
            pltpu.VMEM((half, k), x.dtype),
            pltpu.VMEM((half, k), x.dtype),
            pltpu.VMEM((k, n_loc), jnp.bfloat16),
            pltpu.VMEM((k, wcols), jnp.float32),
            pltpu.VMEM((2, 1, 128), jnp.float32),
            pltpu.VMEM((N_DEV, 1, 128), jnp.float32),
            pltpu.VMEM((N_DEV, 1, 128), jnp.float32),
            pltpu.SemaphoreType.DMA((N_HOPS,)),
            pltpu.SemaphoreType.DMA((N_HOPS,)),
            pltpu.SemaphoreType.DMA((N_HOPS,)),
            pltpu.SemaphoreType.DMA((N_HOPS,)),
            pltpu.SemaphoreType.DMA((2 * (N_DEV - 1),)),
            pltpu.SemaphoreType.DMA((N_DEV,)),
            pltpu.SemaphoreType.DMA((N_DEV,)),
            pltpu.SemaphoreType.DMA,
        ],
        compiler_params=pltpu.CompilerParams(
            collective_id=0, vmem_limit_bytes=64 * 1024 * 1024),
    )(x, w_mat)


# device time: 207467 ns/iter; 1.0025x vs baseline; 1.0025x over previous
import functools

import jax
import jax.numpy as jnp
from jax import lax
from jax.experimental import pallas as pl
from jax.experimental.pallas import tpu as pltpu

N_DEV = 8
N_HOPS = 4


def kernel(x, w_mat):
    x = x.astype(jnp.bfloat16)
    m_per, k = x.shape
    _, n_loc = w_mat.shape
    half = m_per // 2
    wcols = n_loc // 8

    def body(x_ref, w_hbm_ref, out_ref, cw_ref, ccw_ref, cw_half_ref,
             ccw_half_ref, w_ref, w_stage, amax_src, amax_g1, amax_g2,
             cw_send_sems, cw_recv_sems, ccw_send_sems, ccw_recv_sems,
             amax_send_sems, a1_recv_sems, a2_recv_sems, w_sem):
        my = lax.axis_index("i")
        left = lax.rem(my + N_DEV - 1, N_DEV)
        right = lax.rem(my + 1, N_DEV)

        barrier_sem = pltpu.get_barrier_semaphore()
        for nbr in (left, right):
            pl.semaphore_signal(barrier_sem, inc=1, device_id=(nbr,),
                                device_id_type=pl.DeviceIdType.MESH)
        pl.semaphore_wait(barrier_sem, 2)

        def copy(src, dst, ssem, rsem, tgt):
            return pltpu.make_async_remote_copy(
                src_ref=src, dst_ref=dst, send_sem=ssem, recv_sem=rsem,
                device_id=(tgt,), device_id_type=pl.DeviceIdType.MESH)

        def chunk_gemm(chunk, row0, nrows, amax):
            y = jnp.dot(chunk, w_ref[...], preferred_element_type=jnp.float32)
            y = jnp.maximum(y, 0.0)
            out_ref[pl.ds(row0, nrows), :] = y
            return jnp.maximum(amax, jnp.max(y))

        cw = [copy(x_ref, cw_ref.at[0], cw_send_sems.at[0],
                   cw_recv_sems.at[0], right)]
        ccw = [copy(x_ref, ccw_ref.at[0], ccw_send_sems.at[0],
                    ccw_recv_sems.at[0], left)]
        cw[0].start()
        ccw[0].start()

        with jax.named_scope("wconv"):
            for t in range(n_loc // wcols):
                wdma = pltpu.make_async_copy(
                    w_hbm_ref.at[:, pl.ds(t * wcols, wcols)], w_stage, w_sem)
                wdma.start()
                wdma.wait()
                rows = k // 4
                for r in range(4):
                    w_ref[pl.ds(r * rows, rows), pl.ds(t * wcols, wcols)] = (
                        w_stage[pl.ds(r * rows, rows), :].astype(jnp.bfloat16))

        with jax.named_scope("owngemm"):
            amax = chunk_gemm(x_ref[...], my * m_per, m_per, jnp.float32(0.0))

        for h in range(N_HOPS - 1):
            with jax.named_scope(f"waitrecv#hop={h}"):
                cw[h].wait_recv()
            if h < N_HOPS - 2:
                nxt = copy(cw_ref.at[h], cw_ref.at[h + 1],
                           cw_send_sems.at[h + 1], cw_recv_sems.at[h + 1],
                           right)
            else:
                nxt = copy(cw_ref.at[h, pl.ds(0, half)],
                           cw_half_ref,
                           cw_send_sems.at[h + 1], cw_recv_sems.at[h + 1],
                           right)
            nxt.start()
            cw.append(nxt)

            with jax.named_scope(f"waitrecv_ccw#hop={h}"):
                ccw[h].wait_recv()
            if h < N_HOPS - 2:
                nxt = copy(ccw_ref.at[h], ccw_ref.at[h + 1],
                           ccw_send_sems.at[h + 1], ccw_recv_sems.at[h + 1],
                           left)
            else:
                nxt = copy(ccw_ref.at[h, pl.ds(half, half)],
                           ccw_half_ref,
                           ccw_send_sems.at[h + 1], ccw_recv_sems.at[h + 1],
                           left)
            nxt.start()
            ccw.append(nxt)

            with jax.named_scope(f"gemms#hop={h}"):
                cw_origin = lax.rem(my + N_DEV - (h + 1), N_DEV)
                ccw_origin = lax.rem(my + h + 1, N_DEV)
                amax = chunk_gemm(cw_ref[h], cw_origin * m_per, m_per, amax)
                amax = chunk_gemm(ccw_ref[h], ccw_origin * m_per, m_per, amax)

        def amax_exchange(val, src_slot, gather, send_base, recv_sems):
            amax_src[src_slot] = jnp.full((1, 128), val, jnp.float32)
            gather[0] = amax_src[src_slot]
            rs = []
            for d in range(1, N_DEV):
                r = copy(amax_src.at[src_slot], gather.at[d],
                         amax_send_sems.at[send_base + d - 1],
                         recv_sems.at[d], lax.rem(my + d, N_DEV))
                r.start()
                rs.append(r)
            return rs

        def quant_block(row0, nrows, inv, scale):
            blk = out_ref[pl.ds(row0, nrows), :]
            q = (blk * inv).astype(jnp.float8_e4m3fn)
            out_ref[pl.ds(row0, nrows), :] = q.astype(jnp.float32) * scale

        far = lax.rem(my + N_DEV // 2, N_DEV)

        with jax.named_scope("round1"):
            round1 = amax_exchange(amax, 0, amax_g1, 0, a1_recv_sems)
            for r in round1:
                r.wait_recv()
        g1 = jnp.max(amax_g1[...])
        inv1 = 448.0 / g1
        scale1 = g1 / 448.0
        with jax.named_scope("earlyquant"):
            for b in range(N_DEV):
                @pl.when(b != far)
                def _():
                    quant_block(b * m_per, half, inv1, scale1)
                    quant_block(b * m_per + half, half, inv1, scale1)

        with jax.named_scope("hop4wait"):
            cw[N_HOPS - 1].wait_recv()
            ccw[N_HOPS - 1].wait_recv()
        with jax.named_scope("hop4gemm"):
            t1 = chunk_gemm(cw_half_ref[...], far * m_per, half,
                            jnp.float32(0.0))
            tail = chunk_gemm(ccw_half_ref[...], far * m_per + half, half, t1)

        with jax.named_scope("round2"):
            round2 = amax_exchange(tail, 1, amax_g2, N_DEV - 1, a2_recv_sems)
            for r in round2:
                r.wait_recv()
        g2 = jnp.maximum(g1, jnp.max(amax_g2[...]))
        inv2 = 448.0 / g2
        scale2 = g2 / 448.0
        with jax.named_scope("tailquant"):
            quant_block(far * m_per, half, inv2, scale2)
            quant_block(far * m_per + half, half, inv2, scale2)
            for b in range(N_DEV):
                @pl.when(jnp.logical_and(g2 > g1, b != far))
                def _():
                    quant_block(b * m_per, half, inv2, scale2)
                    quant_block(b * m_per + half, half, inv2, scale2)

        with jax.named_scope("drain"):
            for r in cw + ccw + round1 + round2:
                r.wait_send()

        @functools.partial(pl.run_scoped, ack=pltpu.SemaphoreType.REGULAR)
        def _(ack):
            for d in range(1, N_DEV):
                pl.semaphore_signal(ack, inc=1,
                                    device_id=(lax.rem(my + d, N_DEV),),
                                    device_id_type=pl.DeviceIdType.MESH)
            pl.semaphore_wait(ack, N_DEV - 1)

    return pl.pallas_call(
        body,
        out_shape=jax.ShapeDtypeStruct((N_DEV * m_per, n_loc), jnp.float32),
        in_specs=[
            pl.BlockSpec(memory_space=pltpu.VMEM),
            pl.BlockSpec(memory_space=pl.ANY),
        ],
        out_specs=pl.BlockSpec(memory_space=pltpu.VMEM),
        scratch_shapes=[
            pltpu.VMEM((N_HOPS - 1, m_per, k), x.dtype),
            pltpu.VMEM((N_HOPS - 1, m_per, k), x.dtype),
            pltpu.VMEM((half, k), x.dtype),
            pltpu.VMEM((half, k), x.dtype),
            pltpu.VMEM((k, n_loc), jnp.bfloat16),
            pltpu.VMEM((k, wcols), jnp.float32),
            pltpu.VMEM((2, 1, 128), jnp.float32),
            pltpu.VMEM((N_DEV, 1, 128), jnp.float32),
            pltpu.VMEM((N_DEV, 1, 128), jnp.float32),
            pltpu.SemaphoreType.DMA((N_HOPS,)),
            pltpu.SemaphoreType.DMA((N_HOPS,)),
            pltpu.SemaphoreType.DMA((N_HOPS,)),
            pltpu.SemaphoreType.DMA((N_HOPS,)),
            pltpu.SemaphoreType.DMA((2 * (N_DEV - 1),)),
            pltpu.SemaphoreType.DMA((N_DEV,)),
            pltpu.SemaphoreType.DMA((N_DEV,)),
            pltpu.SemaphoreType.DMA,
        ],
        compiler_params=pltpu.CompilerParams(
            collective_id=0, vmem_limit_bytes=64 * 1024 * 1024),
    )(x, w_mat)
